# baseline (speedup 1.0000x reference)
import functools

import jax
import jax.numpy as jnp
from jax import lax
from jax.experimental import pallas as pl
from jax.experimental.pallas import tpu as pltpu
from jax.experimental.pallas import tpu_sc as plsc

_B = 4096
_L = 200
_D = 32
_V = 1000000
_NL = 10
_NW = 32
_BPW = _B // _NW
_C0 = 128
_C1 = _L - _C0
_LP = 208

_CB = 2048
_Q = 1 << 18
_VP = 4 * _Q

_mesh = plsc.VectorSubcoreMesh(core_axis_name="c", subcore_axis_name="s")


def _format_body(in0, in1, in2, in3, out_ref):
    eye = jnp.eye(_D, dtype=jnp.float32)
    for a, ref in enumerate((in0, in1, in2, in3)):
        out_ref[:, pl.ds(32 * a, 32)] = lax.dot_general(
            ref[...], eye, (((0,), (0,)), ((), ())),
            precision=lax.Precision.HIGHEST,
            preferred_element_type=jnp.float32)


def _format_table(tableT):
    q = _Q // _CB
    nbv = (_V - 1) // _CB
    in_specs = [
        pl.BlockSpec((_D, _CB), lambda i, a=a: (0, jnp.minimum(a * q + i, nbv)))
        for a in range(4)
    ]
    return pl.pallas_call(
        _format_body,
        out_shape=jax.ShapeDtypeStruct((_Q, 128), jnp.float32),
        grid=(q,),
        in_specs=in_specs,
        out_specs=pl.BlockSpec((_CB, 128), lambda i: (i, 0)),
    )(tableT, tableT, tableT, tableT)


@functools.partial(
    pl.kernel,
    out_type=jax.ShapeDtypeStruct((_B, _D), jnp.float32),
    mesh=_mesh,
    scratch_types=[
        pltpu.VMEM((_BPW * _L,), jnp.int32),
        pltpu.VMEM((2, _L, _D), jnp.float32),
        pltpu.VMEM((_BPW, _D), jnp.float32),
        pltpu.SemaphoreType.DMA,
        pltpu.SemaphoreType.DMA,
    ],
    compiler_params=pltpu.CompilerParams(
        use_tc_tiling_on_sc=False, needs_layout_passes=False),
)
def _embed_sum(xg_hbm, table_hbm, doc_hbm,
               idx_v, rows_v, doc_v, sem0, sem1):
    wid = lax.axis_index("s") * 2 + lax.axis_index("c")
    base = wid * _BPW
    pltpu.sync_copy(xg_hbm.at[pl.ds(base * _L, _BPW * _L)], idx_v)

    sems = (sem0, sem1)

    def descs(r, p, sem):
        d0 = pltpu.make_async_copy(
            table_hbm.at[idx_v.at[pl.ds(r * _L, _C0)]],
            rows_v.at[p, pl.ds(0, _C0)], sem)
        d1 = pltpu.make_async_copy(
            table_hbm.at[idx_v.at[pl.ds(r * _L + _C0, _C1)]],
            rows_v.at[p, pl.ds(_C0, _C1)], sem)
        return d0, d1

    def issue(r, p):
        d0, d1 = descs(r, p, sems[p])
        d0.start()
        d1.start()

    issue(0, 0)
    issue(1, 1)

    zeros = jnp.zeros((16,), jnp.float32)

    def outer(g, carry):
        for p in range(2):
            r = g * 2 + p
            d0, d1 = descs(r, p, sems[p])
            d0.wait()
            d1.wait()

            def rbody(jj, acc):
                a0, a1, b0, b1 = acc
                for u in range(8):
                    j = jj * 8 + u
                    lo = rows_v[p, j, pl.ds(0, 16)]
                    hi = rows_v[p, j, pl.ds(16, 16)]
                    if u % 2 == 0:
                        a0 = a0 + lo
                        a1 = a1 + hi
                    else:
                        b0 = b0 + lo
                        b1 = b1 + hi
                return (a0, a1, b0, b1)

            a0, a1, b0, b1 = lax.fori_loop(
                0, _L // 8, rbody, (zeros, zeros, zeros, zeros))

            @pl.when(r + 2 < _BPW)
            def _():
                issue(r + 2, p)

            doc_v[r, pl.ds(0, 16)] = a0 + b0
            doc_v[r, pl.ds(16, 16)] = a1 + b1
        return carry

    lax.fori_loop(0, _BPW // 2, outer, 0)
    pltpu.sync_copy(doc_v, doc_hbm.at[pl.ds(base, _BPW)])


def _head_body(doc_ref, w_ref, b_ref, out_ref):
    out_ref[...] = (
        jnp.dot(doc_ref[...], w_ref[...], preferred_element_type=jnp.float32)
        + b_ref[...]
    )


def _head(doc, W, b):
    nblk = 4
    return pl.pallas_call(
        _head_body,
        out_shape=jax.ShapeDtypeStruct((_B, _NL), jnp.float32),
        grid=(nblk,),
        in_specs=[
            pl.BlockSpec((_B // nblk, _D), lambda i: (i, 0)),
            pl.BlockSpec((_D, _NL), lambda i: (0, 0)),
            pl.BlockSpec((1, _NL), lambda i: (0, 0)),
        ],
        out_specs=pl.BlockSpec((_B // nblk, _NL), lambda i: (i, 0)),
    )(doc, W, b.reshape(1, _NL))


def kernel(x, m, table, W, b):
    del m
    x = x.astype(jnp.int32)
    xf = (((x & (_Q - 1)) << 2) | (x >> 18)).reshape(-1)
    t2 = _format_table(table.T)
    doc = _embed_sum(xf, t2.reshape(_VP, _D))
    return _head(doc, W, b)


# speedup vs baseline: 1.8130x; 1.8130x over previous
import functools

import jax
import jax.numpy as jnp
from jax import lax
from jax.experimental import pallas as pl
from jax.experimental.pallas import tpu as pltpu
from jax.experimental.pallas import tpu_sc as plsc

_B = 4096
_L = 200
_D = 32
_V = 1000000
_NL = 10
_NW = 32
_BPW = _B // _NW
_C0 = 128
_C1 = _L - _C0
_LP = 208

_CB = 8192
_Q = 1 << 18
_VP = 4 * _Q

_mesh = plsc.VectorSubcoreMesh(core_axis_name="c", subcore_axis_name="s")


def _format_body(in0, in1, in2, in3, out_ref):
    for a, ref in enumerate((in0, in1, in2, in3)):
        out_ref[:, pl.ds(32 * a, 32)] = ref[...].T


def _format_table(tableT):
    q = _Q // _CB
    nbv = (_V - 1) // _CB
    in_specs = [
        pl.BlockSpec((_D, _CB), lambda i, a=a: (0, jnp.minimum(a * q + i, nbv)))
        for a in range(4)
    ]
    return pl.pallas_call(
        _format_body,
        out_shape=jax.ShapeDtypeStruct((_Q, 128), jnp.float32),
        grid=(q,),
        in_specs=in_specs,
        out_specs=pl.BlockSpec((_CB, 128), lambda i: (i, 0)),
    )(tableT, tableT, tableT, tableT)


@functools.partial(
    pl.kernel,
    out_type=jax.ShapeDtypeStruct((_B, _D), jnp.float32),
    mesh=_mesh,
    scratch_types=[
        pltpu.VMEM((_BPW * _L,), jnp.int32),
        pltpu.VMEM((2, _L, _D), jnp.float32),
        pltpu.VMEM((_BPW, _D), jnp.float32),
        pltpu.SemaphoreType.DMA,
        pltpu.SemaphoreType.DMA,
    ],
    compiler_params=pltpu.CompilerParams(
        use_tc_tiling_on_sc=False, needs_layout_passes=False),
)
def _embed_sum(xg_hbm, table_hbm, doc_hbm,
               idx_v, rows_v, doc_v, sem0, sem1):
    wid = lax.axis_index("s") * 2 + lax.axis_index("c")
    base = wid * _BPW
    pltpu.sync_copy(xg_hbm.at[pl.ds(base * _L, _BPW * _L)], idx_v)

    sems = (sem0, sem1)

    def descs(r, p, sem):
        d0 = pltpu.make_async_copy(
            table_hbm.at[idx_v.at[pl.ds(r * _L, _C0)]],
            rows_v.at[p, pl.ds(0, _C0)], sem)
        d1 = pltpu.make_async_copy(
            table_hbm.at[idx_v.at[pl.ds(r * _L + _C0, _C1)]],
            rows_v.at[p, pl.ds(_C0, _C1)], sem)
        return d0, d1

    def issue(r, p):
        d0, d1 = descs(r, p, sems[p])
        d0.start()
        d1.start()

    issue(0, 0)
    issue(1, 1)

    zeros = jnp.zeros((16,), jnp.float32)

    def outer(g, carry):
        for p in range(2):
            r = g * 2 + p
            d0, d1 = descs(r, p, sems[p])
            d0.wait()
            d1.wait()

            def rbody(jj, acc):
                a0, a1, b0, b1 = acc
                for u in range(8):
                    j = jj * 8 + u
                    lo = rows_v[p, j, pl.ds(0, 16)]
                    hi = rows_v[p, j, pl.ds(16, 16)]
                    if u % 2 == 0:
                        a0 = a0 + lo
                        a1 = a1 + hi
                    else:
                        b0 = b0 + lo
                        b1 = b1 + hi
                return (a0, a1, b0, b1)

            a0, a1, b0, b1 = lax.fori_loop(
                0, _L // 8, rbody, (zeros, zeros, zeros, zeros))

            @pl.when(r + 2 < _BPW)
            def _():
                issue(r + 2, p)

            doc_v[r, pl.ds(0, 16)] = a0 + b0
            doc_v[r, pl.ds(16, 16)] = a1 + b1
        return carry

    lax.fori_loop(0, _BPW // 2, outer, 0)
    pltpu.sync_copy(doc_v, doc_hbm.at[pl.ds(base, _BPW)])


def _head_body(doc_ref, w_ref, b_ref, out_ref):
    out_ref[...] = (
        jnp.dot(doc_ref[...], w_ref[...], preferred_element_type=jnp.float32)
        + b_ref[...]
    )


def _head(doc, W, b):
    nblk = 4
    return pl.pallas_call(
        _head_body,
        out_shape=jax.ShapeDtypeStruct((_B, _NL), jnp.float32),
        grid=(nblk,),
        in_specs=[
            pl.BlockSpec((_B // nblk, _D), lambda i: (i, 0)),
            pl.BlockSpec((_D, _NL), lambda i: (0, 0)),
            pl.BlockSpec((1, _NL), lambda i: (0, 0)),
        ],
        out_specs=pl.BlockSpec((_B // nblk, _NL), lambda i: (i, 0)),
    )(doc, W, b.reshape(1, _NL))


def kernel(x, m, table, W, b):
    del m
    x = x.astype(jnp.int32)
    xf = (((x & (_Q - 1)) << 2) | (x >> 18)).reshape(-1)
    t2 = _format_table(table.T)
    doc = _embed_sum(xf, t2.reshape(_VP, _D))
    return _head(doc, W, b)


# speedup vs baseline: 3.2682x; 1.8027x over previous
import functools

import jax
import jax.numpy as jnp
from jax import lax
from jax.experimental import pallas as pl
from jax.experimental.pallas import tpu as pltpu
from jax.experimental.pallas import tpu_sc as plsc

_B = 4096
_L = 200
_D = 32
_V = 1000000
_NL = 10
_NW = 32
_BPW = _B // _NW
_C0 = 128
_C1 = _L - _C0
_LP = 208

_CB = 8192
_Q = 1 << 18
_VP = 4 * _Q

_mesh = plsc.VectorSubcoreMesh(core_axis_name="c", subcore_axis_name="s")


def _format_body(in0, in1, in2, in3, out_ref):
    blk = jnp.concatenate(
        [in0[...], in1[...], in2[...], in3[...]], axis=0)
    out_ref[...] = blk.T


def _format_table(tableT):
    q = _Q // _CB
    nbv = (_V - 1) // _CB
    in_specs = [
        pl.BlockSpec((_D, _CB), lambda i, a=a: (0, jnp.minimum(a * q + i, nbv)))
        for a in range(4)
    ]
    return pl.pallas_call(
        _format_body,
        out_shape=jax.ShapeDtypeStruct((_Q, 128), jnp.float32),
        grid=(q,),
        in_specs=in_specs,
        out_specs=pl.BlockSpec((_CB, 128), lambda i: (i, 0)),
    )(tableT, tableT, tableT, tableT)


@functools.partial(
    pl.kernel,
    out_type=jax.ShapeDtypeStruct((_B, _D), jnp.float32),
    mesh=_mesh,
    scratch_types=[
        pltpu.VMEM((_BPW * _L,), jnp.int32),
        pltpu.VMEM((2, _L, _D), jnp.float32),
        pltpu.VMEM((_BPW, _D), jnp.float32),
        pltpu.SemaphoreType.DMA,
        pltpu.SemaphoreType.DMA,
    ],
    compiler_params=pltpu.CompilerParams(
        use_tc_tiling_on_sc=False, needs_layout_passes=False),
)
def _embed_sum(xg_hbm, table_hbm, doc_hbm,
               idx_v, rows_v, doc_v, sem0, sem1):
    wid = lax.axis_index("s") * 2 + lax.axis_index("c")
    base = wid * _BPW
    pltpu.sync_copy(xg_hbm.at[pl.ds(base * _L, _BPW * _L)], idx_v)

    sems = (sem0, sem1)

    def descs(r, p, sem):
        d0 = pltpu.make_async_copy(
            table_hbm.at[idx_v.at[pl.ds(r * _L, _C0)]],
            rows_v.at[p, pl.ds(0, _C0)], sem)
        d1 = pltpu.make_async_copy(
            table_hbm.at[idx_v.at[pl.ds(r * _L + _C0, _C1)]],
            rows_v.at[p, pl.ds(_C0, _C1)], sem)
        return d0, d1

    def issue(r, p):
        d0, d1 = descs(r, p, sems[p])
        d0.start()
        d1.start()

    issue(0, 0)
    issue(1, 1)

    zeros = jnp.zeros((16,), jnp.float32)

    def outer(g, carry):
        for p in range(2):
            r = g * 2 + p
            d0, d1 = descs(r, p, sems[p])
            d0.wait()
            d1.wait()

            def rbody(jj, acc):
                a0, a1, b0, b1 = acc
                for u in range(8):
                    j = jj * 8 + u
                    lo = rows_v[p, j, pl.ds(0, 16)]
                    hi = rows_v[p, j, pl.ds(16, 16)]
                    if u % 2 == 0:
                        a0 = a0 + lo
                        a1 = a1 + hi
                    else:
                        b0 = b0 + lo
                        b1 = b1 + hi
                return (a0, a1, b0, b1)

            a0, a1, b0, b1 = lax.fori_loop(
                0, _L // 8, rbody, (zeros, zeros, zeros, zeros))

            @pl.when(r + 2 < _BPW)
            def _():
                issue(r + 2, p)

            doc_v[r, pl.ds(0, 16)] = a0 + b0
            doc_v[r, pl.ds(16, 16)] = a1 + b1
        return carry

    lax.fori_loop(0, _BPW // 2, outer, 0)
    pltpu.sync_copy(doc_v, doc_hbm.at[pl.ds(base, _BPW)])


def _head_body(doc_ref, w_ref, b_ref, out_ref):
    out_ref[...] = (
        jnp.dot(doc_ref[...], w_ref[...], preferred_element_type=jnp.float32)
        + b_ref[...]
    )


def _head(doc, W, b):
    nblk = 4
    return pl.pallas_call(
        _head_body,
        out_shape=jax.ShapeDtypeStruct((_B, _NL), jnp.float32),
        grid=(nblk,),
        in_specs=[
            pl.BlockSpec((_B // nblk, _D), lambda i: (i, 0)),
            pl.BlockSpec((_D, _NL), lambda i: (0, 0)),
            pl.BlockSpec((1, _NL), lambda i: (0, 0)),
        ],
        out_specs=pl.BlockSpec((_B // nblk, _NL), lambda i: (i, 0)),
    )(doc, W, b.reshape(1, _NL))


def kernel(x, m, table, W, b):
    del m
    x = x.astype(jnp.int32)
    xf = (((x & (_Q - 1)) << 2) | (x >> 18)).reshape(-1)
    t2 = _format_table(table.T)
    doc = _embed_sum(xf, t2.reshape(_VP, _D))
    return _head(doc, W, b)
